# baseline (speedup 1.0000x reference)
import functools

import jax
import jax.numpy as jnp
from jax import lax
from jax.experimental import pallas as pl
from jax.experimental.pallas import tpu as pltpu
from jax.experimental.pallas import tpu_sc as plsc

N = 19
E = 342
D_IN = 1025
HID = 256
OUT = 512

LANES = 16
NCHUNK = (E + LANES - 1) // LANES
E_PAD = NCHUNK * LANES
TAIL = E - (NCHUNK - 1) * LANES
A_COLS = 32


def _sc_build_adjacency(ei_hbm, w_hbm, wp_hbm, a_hbm, wp2_hbm,
                        ei_v, w_v, a_v, wp_v, wpf_v, sem, sem2):
    cid = lax.axis_index("c")
    sid = lax.axis_index("s")

    @pl.when(jnp.logical_and(cid == 1, sid == 0))
    def _():
        pltpu.sync_copy(wp_hbm, wpf_v)
        for d in range(N):
            for k in range(OUT // LANES):
                wp_v[d, pl.ds(k * LANES, LANES)] = (
                    wpf_v[pl.ds(d * OUT + k * LANES, LANES)])
        pltpu.sync_copy(wp_v, wp2_hbm)

    @pl.when(jnp.logical_and(cid == 0, sid == 0))
    def _():
        c1 = pltpu.make_async_copy(ei_hbm, ei_v, sem)
        c2 = pltpu.make_async_copy(w_hbm, w_v, sem)
        c1.start()
        c2.start()
        zero = jnp.zeros((LANES,), jnp.float32)
        for d in range(N):
            a_v[d, pl.ds(0, LANES)] = zero
            a_v[d, pl.ds(LANES, LANES)] = zero
        c1.wait()
        c2.wait()
        lane = lax.iota(jnp.int32, LANES)
        offs = [e * LANES for e in range(NCHUNK - 1)] + [E - LANES]
        for i, off in enumerate(offs):
            s = ei_v[0, pl.ds(off, LANES)]
            d = ei_v[1, pl.ds(off, LANES)]
            w = w_v[pl.ds(off, LANES)]
            if i == NCHUNK - 1:
                plsc.addupdate_scatter(a_v, [d, s], w,
                                       mask=lane >= (LANES - TAIL))
            else:
                plsc.addupdate_scatter(a_v, [d, s], w)
        pltpu.sync_copy(a_v, a_hbm)


def _sc_adjacency_call(edge_index, w, wp):
    run = functools.partial(
        pl.kernel,
        out_type=(
            jax.ShapeDtypeStruct((N, A_COLS), jnp.float32),
            jax.ShapeDtypeStruct((N, OUT), jnp.float32),
        ),
        mesh=plsc.VectorSubcoreMesh(core_axis_name="c", subcore_axis_name="s"),
        scratch_types=[
            pltpu.VMEM((2, E), jnp.int32),
            pltpu.VMEM((E,), jnp.float32),
            pltpu.VMEM((N, A_COLS), jnp.float32),
            pltpu.VMEM((N, OUT), jnp.float32),
            pltpu.VMEM((N * OUT,), jnp.float32),
            pltpu.SemaphoreType.DMA,
            pltpu.SemaphoreType.DMA,
        ],
        compiler_params=pltpu.CompilerParams(needs_layout_passes=False),
    )(_sc_build_adjacency)
    return run(edge_index, w, wp)


def _tc_first_mm(x_ref, w1a_ref, y_ref):
    y_ref[...] = jnp.dot(x_ref[...], w1a_ref[...],
                         preferred_element_type=jnp.float32)


def _tc_dense(a_ref, y1_ref, b1a_ref, w1b_ref, b1b_ref,
              w2a_ref, b2a_ref, w2b_ref, b2b_ref, wp_ref, bp_ref, out_ref):
    a = lax.slice(a_ref[...], (0, 0), (N, N))
    y1 = y1_ref[...]
    t = jnp.maximum(
        y1 + jnp.dot(a, y1, preferred_element_type=jnp.float32)
        + b1a_ref[...][None, :], 0.0)
    h1 = jnp.maximum(
        jnp.dot(t, w1b_ref[...], preferred_element_type=jnp.float32)
        + b1b_ref[...][None, :], 0.0)
    z2 = h1 + jnp.dot(a, h1, preferred_element_type=jnp.float32)
    u = jnp.maximum(
        jnp.dot(z2, w2a_ref[...], preferred_element_type=jnp.float32)
        + b2a_ref[...][None, :], 0.0)
    h2 = (jnp.dot(u, w2b_ref[...], preferred_element_type=jnp.float32)
          + b2b_ref[...][None, :])
    s = jnp.sum(h2 * wp_ref[...], keepdims=True) + bp_ref[...][None, :]
    out_ref[...] = 1.0 / (1.0 + jnp.exp(-s))


@jax.jit
def kernel(x, edge_index, pred_connectivity,
           W1a, b1a, W1b, b1b, W2a, b2a, W2b, b2b, Wp, bp):
    a, wp2 = _sc_adjacency_call(edge_index, pred_connectivity,
                                Wp.reshape(N * OUT))

    y1 = pl.pallas_call(
        _tc_first_mm,
        out_shape=jax.ShapeDtypeStruct((N, HID), jnp.float32),
    )(x, W1a)

    out = pl.pallas_call(
        _tc_dense,
        out_shape=jax.ShapeDtypeStruct((1, 1), jnp.float32),
    )(
        a, y1,
        b1a, W1b, b1b,
        W2a, b2a, W2b, b2b,
        wp2, bp,
    )
    return out.reshape(1)


# speedup vs baseline: 1.1596x; 1.1596x over previous
import functools

import jax
import jax.numpy as jnp
from jax import lax
from jax.experimental import pallas as pl
from jax.experimental.pallas import tpu as pltpu
from jax.experimental.pallas import tpu_sc as plsc

N = 19
E = 342
D_IN = 1025
HID = 256
OUT = 512

LANES = 16
NCHUNK = (E + LANES - 1) // LANES
E_PAD = NCHUNK * LANES
TAIL = E - (NCHUNK - 1) * LANES
A_COLS = 32


def _sc_build_adjacency(ei_hbm, w_hbm, a_hbm, ei_v, w_v, a_v, sem):
    c1 = pltpu.make_async_copy(ei_hbm, ei_v, sem)
    c2 = pltpu.make_async_copy(w_hbm, w_v, sem)
    c1.start()
    c2.start()
    zero = jnp.zeros((LANES,), jnp.float32)
    for d in range(N):
        a_v[d, pl.ds(0, LANES)] = zero
        a_v[d, pl.ds(LANES, LANES)] = zero
    c1.wait()
    c2.wait()

    def chunk(i, carry):
        off = pl.multiple_of(i * LANES, LANES)
        s = ei_v[0, pl.ds(off, LANES)]
        d = ei_v[1, pl.ds(off, LANES)]
        w = w_v[pl.ds(off, LANES)]
        plsc.addupdate_scatter(a_v, [d, s], w)
        return carry

    lax.fori_loop(0, NCHUNK - 1, chunk, 0)
    lane = lax.iota(jnp.int32, LANES)
    s = ei_v[0, pl.ds(E - LANES, LANES)]
    d = ei_v[1, pl.ds(E - LANES, LANES)]
    w = w_v[pl.ds(E - LANES, LANES)]
    plsc.addupdate_scatter(a_v, [d, s], w, mask=lane >= (LANES - TAIL))
    pltpu.sync_copy(a_v, a_hbm)


def _sc_adjacency_call(edge_index, w):
    run = functools.partial(
        pl.kernel,
        out_type=jax.ShapeDtypeStruct((N, A_COLS), jnp.float32),
        mesh=plsc.VectorSubcoreMesh(core_axis_name="c", subcore_axis_name="s",
                                    num_cores=1, num_subcores=1),
        scratch_types=[
            pltpu.VMEM((2, E), jnp.int32),
            pltpu.VMEM((E,), jnp.float32),
            pltpu.VMEM((N, A_COLS), jnp.float32),
            pltpu.SemaphoreType.DMA,
        ],
        compiler_params=pltpu.CompilerParams(needs_layout_passes=False),
    )(_sc_build_adjacency)
    return run(edge_index, w)


def _tc_first_mm(x_ref, w1a_ref, y_ref):
    y_ref[...] = jnp.dot(x_ref[...], w1a_ref[...],
                         preferred_element_type=jnp.float32)


def _tc_dense(a_ref, y1_ref, b1a_ref, w1b_ref, b1b_ref,
              w2a_ref, b2a_ref, w2b_ref, b2b_ref, wp_ref, bp_ref, out_ref):
    a = lax.slice(a_ref[...], (0, 0), (N, N))
    y1 = y1_ref[...]
    t = jnp.maximum(
        y1 + jnp.dot(a, y1, preferred_element_type=jnp.float32)
        + b1a_ref[...][None, :], 0.0)
    h1 = jnp.maximum(
        jnp.dot(t, w1b_ref[...], preferred_element_type=jnp.float32)
        + b1b_ref[...][None, :], 0.0)
    z2 = h1 + jnp.dot(a, h1, preferred_element_type=jnp.float32)
    u = jnp.maximum(
        jnp.dot(z2, w2a_ref[...], preferred_element_type=jnp.float32)
        + b2a_ref[...][None, :], 0.0)
    h2 = (jnp.dot(u, w2b_ref[...], preferred_element_type=jnp.float32)
          + b2b_ref[...][None, :])
    s = jnp.sum(h2 * wp_ref[...], keepdims=True) + bp_ref[...][None, :]
    out_ref[...] = 1.0 / (1.0 + jnp.exp(-s))


@jax.jit
def kernel(x, edge_index, pred_connectivity,
           W1a, b1a, W1b, b1b, W2a, b2a, W2b, b2b, Wp, bp):
    a = _sc_adjacency_call(edge_index, pred_connectivity)

    y1 = pl.pallas_call(
        _tc_first_mm,
        out_shape=jax.ShapeDtypeStruct((N, HID), jnp.float32),
    )(x, W1a)

    out = pl.pallas_call(
        _tc_dense,
        out_shape=jax.ShapeDtypeStruct((1, 1), jnp.float32),
    )(
        a, y1,
        b1a, W1b, b1b,
        W2a, b2a, W2b, b2b,
        Wp.reshape(N, OUT), bp,
    )
    return out.reshape(1)
